# baseline (speedup 1.0000x reference)
import functools

import jax
import jax.numpy as jnp
from jax import lax
from jax.experimental import pallas as pl
from jax.experimental.pallas import tpu as pltpu

_DROPOUT_P = 0.5
_DROPOUT_SEED = 1234
_GOLDEN = 0x9E3779B9


def _fused_kernel(x_ref, w_ref, b_ref, o_ref, xb_ref, lin_ref, *, n_total,
                  seed_u, scale):
    j = pl.program_id(1)
    tm, tn = o_ref.shape
    first = jnp.logical_and(pl.program_id(0) == 0, j == 0)

    @pl.when(first)
    def _():
        lin_ref[...] = (lax.broadcasted_iota(jnp.int32, (tm, tn), 0) * n_total
                        + lax.broadcasted_iota(jnp.int32, (tm, tn), 1)
                        ).astype(jnp.uint32)

    @pl.when(j == 0)
    def _():
        xb_ref[...] = x_ref[...].astype(jnp.bfloat16)

    base = ((pl.program_id(0) * tm) * n_total + j * tn).astype(jnp.uint32)
    xb = xb_ref[...]
    half = tn // 2
    h = (lin_ref[...] + base) ^ jnp.uint32(seed_u)
    h = h ^ (h >> 16)
    h = h * jnp.uint32(0x85EBCA6B)
    h = h ^ (h >> 13)
    h = h * jnp.uint32(0xC2B2AE35)
    m = jnp.where((h & jnp.uint32(0x00800000)) != 0, jnp.float32(scale), 0.0)
    for h0 in range(2):
        lo, hi = h0 * half, (h0 + 1) * half
        acc = lax.dot_general(
            xb, w_ref[lo:hi, :].astype(jnp.bfloat16),
            dimension_numbers=(((1,), (1,)), ((), ())),
            preferred_element_type=jnp.float32)
        o_ref[:, lo:hi] = jnp.maximum(acc + b_ref[:, lo:hi], 0.0) * m[:, lo:hi]


def kernel(x, w, b):
    B, K = x.shape
    N, Kw = w.shape
    assert Kw == K

    bm = min(1024, B)
    bn = min(512, N)
    grid = (B // bm, N // bn)

    b2 = b.reshape(1, N).astype(jnp.float32)

    seed_u = (_DROPOUT_SEED * _GOLDEN) & 0xFFFFFFFF
    body = functools.partial(
        _fused_kernel, n_total=N, seed_u=seed_u,
        scale=1.0 / (1.0 - _DROPOUT_P))

    out = pl.pallas_call(
        body,
        grid=grid,
        in_specs=[
            pl.BlockSpec((bm, K), lambda i, j: (i, 0)),
            pl.BlockSpec((bn, K), lambda i, j: (j, 0)),
            pl.BlockSpec((1, bn), lambda i, j: (0, j)),
        ],
        out_specs=pl.BlockSpec((bm, bn), lambda i, j: (i, j)),
        out_shape=jax.ShapeDtypeStruct((B, N), jnp.float32),
        scratch_shapes=[
            pltpu.VMEM((bm, K), jnp.bfloat16),
            pltpu.VMEM((bm, bn), jnp.uint32),
        ],
        compiler_params=pltpu.CompilerParams(
            dimension_semantics=("arbitrary", "arbitrary"),
            vmem_limit_bytes=64 * 1024 * 1024),
    )(x, w, b2)
    return out
